# baseline (speedup 1.0000x reference)
import functools

import jax
import jax.numpy as jnp
import numpy as np
from jax import lax
from jax.experimental import pallas as pl
from jax.experimental.pallas import tpu as pltpu
from jax.experimental.pallas import tpu_sc as plsc

N_NODES = 10000
NP = 10240
NA = 10112
D = 128
NTILES = 32
ROWS_PER_TILE = NP // 16
CHUNK = 128
GRP = 8
ROWBLK = 1280

f32 = jnp.float32
bf16 = jnp.bfloat16

_PERM = np.empty((D,), np.int64)
for _g in range(D // 32):
    for _j in range(16):
        _PERM[_g * 32 + 2 * _j] = _g * 32 + _j
        _PERM[_g * 32 + 2 * _j + 1] = _g * 32 + 16 + _j


def _mm_att_body(x_ref, w_ref, am_ref, h_ref, aa_ref):
    h = jnp.dot(x_ref[...], w_ref[...], preferred_element_type=f32)
    h_ref[...] = h.astype(bf16)
    aa_ref[...] = jnp.dot(h, am_ref[...], preferred_element_type=f32)


def _mm_att(xp, W, attm):
    grid = (NP // ROWBLK,)
    return pl.pallas_call(
        _mm_att_body,
        grid=grid,
        in_specs=[
            pl.BlockSpec((ROWBLK, D), lambda i: (i, 0)),
            pl.BlockSpec((D, D), lambda i: (0, 0)),
            pl.BlockSpec((D, D), lambda i: (0, 0)),
        ],
        out_specs=[
            pl.BlockSpec((ROWBLK, D), lambda i: (i, 0)),
            pl.BlockSpec((ROWBLK, D), lambda i: (i, 0)),
        ],
        out_shape=[
            jax.ShapeDtypeStruct((NP, D), bf16),
            jax.ShapeDtypeStruct((NP, D), f32),
        ],
    )(xp, W, attm)


def _combine_mm_body(n_ref, d_ref, b_ref, w_ref, am_ref, h_ref, aa_ref):
    num = n_ref[0] + n_ref[1]
    den = jnp.sum(d_ref[...], axis=0) + 1e-16
    o = num / den[:, None] + b_ref[...]
    h_in = jnp.maximum(o, 0.0)
    h = jnp.dot(h_in, w_ref[...], preferred_element_type=f32)
    h_ref[...] = h.astype(bf16)
    aa_ref[...] = jnp.dot(h, am_ref[...], preferred_element_type=f32)


def _combine_mm(nacc, dparts, b2d, W, attm):
    grid = (NP // ROWBLK,)
    return pl.pallas_call(
        _combine_mm_body,
        grid=grid,
        in_specs=[
            pl.BlockSpec((2, ROWBLK, D), lambda i: (0, i, 0)),
            pl.BlockSpec((2, ROWBLK), lambda i: (0, i)),
            pl.BlockSpec((1, D), lambda i: (0, 0)),
            pl.BlockSpec((D, D), lambda i: (0, 0)),
            pl.BlockSpec((D, D), lambda i: (0, 0)),
        ],
        out_specs=[
            pl.BlockSpec((ROWBLK, D), lambda i: (i, 0)),
            pl.BlockSpec((ROWBLK, D), lambda i: (i, 0)),
        ],
        out_shape=[
            jax.ShapeDtypeStruct((NP, D), bf16),
            jax.ShapeDtypeStruct((NP, D), f32),
        ],
    )(nacc, dparts, b2d, W, attm)


def _combine_last_body(n_ref, d_ref, b_ref, o_ref):
    num = n_ref[0] + n_ref[1]
    den = jnp.sum(d_ref[...], axis=0) + 1e-16
    o_ref[...] = num / den[:, None] + b_ref[...]


def _combine_last(nacc, dparts, b2d):
    grid = (NP // ROWBLK,)
    return pl.pallas_call(
        _combine_last_body,
        grid=grid,
        in_specs=[
            pl.BlockSpec((2, ROWBLK, D), lambda i: (0, i, 0)),
            pl.BlockSpec((2, ROWBLK), lambda i: (0, i)),
            pl.BlockSpec((1, D), lambda i: (0, 0)),
        ],
        out_specs=pl.BlockSpec((ROWBLK, D), lambda i: (i, 0)),
        out_shape=jax.ShapeDtypeStruct((NP, D), f32),
    )(nacc, dparts, b2d)


def _make_edge_pass(cpt0, cpt1):
    cpt = max(cpt0, cpt1)
    mesh = plsc.VectorSubcoreMesh(core_axis_name="c", subcore_axis_name="s")

    @functools.partial(
        pl.kernel,
        out_type=(
            jax.ShapeDtypeStruct((2, NP, D), f32),
            jax.ShapeDtypeStruct((2, NP), f32),
        ),
        mesh=mesh,
        compiler_params=pltpu.CompilerParams(
            needs_layout_passes=False, use_tc_tiling_on_sc=False),
        scratch_types=(
            pltpu.VMEM((2 * GRP, CHUNK), jnp.int32),
            pltpu.VMEM((2 * GRP, CHUNK), jnp.int32),
            pltpu.VMEM((2 * CHUNK, D), bf16),
            pltpu.VMEM((1, CHUNK), f32),
            pltpu.VMEM((1, CHUNK), f32),
            pltpu.VMEM((CHUNK, D), f32),
            pltpu.VMEM((CHUNK,), f32),
            pltpu.VMEM((ROWS_PER_TILE,), f32),
            pltpu.VMEM_SHARED((NP, D), f32),
            pltpu.VMEM_SHARED((NP,), f32),
            pltpu.VMEM_SHARED((NA,), f32),
            pltpu.VMEM_SHARED((NA,), f32),
            pltpu.SemaphoreType.DMA,
            pltpu.SemaphoreType.DMA,
        ),
    )
    def edge_pass(h_hbm, as_hbm, ad_hbm, src_hbm, dst_hbm,
                  nacc_hbm, dden_hbm,
                  srcl, dstl, rowsb, asr, adr, rows, epb, zb,
                  nsh, dsh, ash, adh, sem0, sem1):
        c = lax.axis_index("c")
        s = lax.axis_index("s")
        wid = c * 16 + s
        cpt_c = jnp.where(c == 0, cpt0, cpt1)
        ngrp = cpt_c // GRP

        na16 = NA // 16
        pltpu.sync_copy(as_hbm.at[pl.ds(s * na16, na16)],
                        ash.at[pl.ds(s * na16, na16)])
        pltpu.sync_copy(ad_hbm.at[pl.ds(s * na16, na16)],
                        adh.at[pl.ds(s * na16, na16)])

        zv = jnp.zeros((16,), f32)

        def zero_rows(r, _):
            for k in range(8):
                rows[r, pl.ds(16 * k, 16)] = zv
            return _
        lax.fori_loop(0, CHUNK, zero_rows, None)

        def zero_zb(i, _):
            zb[pl.ds(16 * i, 16)] = zv
            return _
        lax.fori_loop(0, ROWS_PER_TILE // 16, zero_zb, None)

        base = s * ROWS_PER_TILE
        for k in range(ROWS_PER_TILE // CHUNK):
            pltpu.sync_copy(rows, nsh.at[pl.ds(base + k * CHUNK, CHUNK)])
        pltpu.sync_copy(zb, dsh.at[pl.ds(base, ROWS_PER_TILE)])
        plsc.subcore_barrier()

        def stage_group(g):
            gp = lax.rem(g, 2)
            off = pl.multiple_of(g * GRP, GRP)
            dstrow = pl.multiple_of(gp * GRP, GRP)
            pltpu.sync_copy(src_hbm.at[wid, pl.ds(off, GRP)],
                            srcl.at[pl.ds(dstrow, GRP)])
            pltpu.sync_copy(dst_hbm.at[wid, pl.ds(off, GRP)],
                            dstl.at[pl.ds(dstrow, GRP)])

        def row_of(j):
            return lax.rem(j // GRP, 2) * GRP + lax.rem(j, GRP)

        def fire(j):
            p = lax.rem(j, 2)
            row = row_of(j)
            sm = [sem0, sem1]
            for pp in range(2):
                @pl.when(p == pp)
                def _():
                    off = pl.multiple_of(pp * CHUNK, CHUNK)
                    pltpu.async_copy(h_hbm.at[srcl.at[row]],
                                     rowsb.at[pl.ds(off, CHUNK)], sm[pp])

        def drain(j):
            p = lax.rem(j, 2)
            row = row_of(j)
            sm = [sem0, sem1]
            for pp in range(2):
                @pl.when(p == pp)
                def _():
                    off = pl.multiple_of(pp * CHUNK, CHUNK)
                    pltpu.make_async_copy(
                        h_hbm.at[srcl.at[row]],
                        rowsb.at[pl.ds(off, CHUNK)], sm[pp]).wait()

        stage_group(0)
        fire(0)

        def chunk_body(j, _):
            p = lax.rem(j, 2)
            row = row_of(j)

            g = j // GRP
            @pl.when(jnp.logical_and(lax.rem(j, GRP) == 0, g + 1 < ngrp))
            def _stage():
                stage_group(g + 1)

            drain(j)

            @pl.when(j + 1 < cpt_c)
            def _fire():
                fire(j + 1)

            pltpu.sync_copy(ash.at[srcl.at[row]], asr.at[0])
            pltpu.sync_copy(adh.at[dstl.at[row]], adr.at[0])

            for i in range(CHUNK // 16):
                a = asr[0, pl.ds(16 * i, 16)]
                b = adr[0, pl.ds(16 * i, 16)]
                e = a + b
                e = jnp.where(e >= 0.0, e, 0.2 * e)
                w = jnp.exp(e)
                epb[pl.ds(16 * i, 16)] = w

            pltpu.sync_copy(epb, dsh.at[dstl.at[row]], add=True)

            rb = p * CHUNK

            def scale16(i2, _):
                off = pl.multiple_of(i2 * 16, 16)
                wv = epb[pl.ds(off, 16)]
                r0 = rb + i2 * 16
                for l in range(16):
                    sc = wv[l]
                    for k in range(4):
                        packed = plsc.bitcast(
                            rowsb[r0 + l, pl.ds(32 * k, 32)], jnp.int32)
                        lo = plsc.bitcast(packed << 16, f32)
                        hi = plsc.bitcast(packed & jnp.int32(-65536), f32)
                        rows[i2 * 16 + l, pl.ds(32 * k, 16)] = lo * sc
                        rows[i2 * 16 + l, pl.ds(32 * k + 16, 16)] = hi * sc
                return _
            lax.fori_loop(0, CHUNK // 16, scale16, None)

            pltpu.sync_copy(rows, nsh.at[dstl.at[row]], add=True)
            return _

        lax.fori_loop(0, cpt_c, chunk_body, None)
        plsc.subcore_barrier()

        pltpu.sync_copy(
            nsh.at[pl.ds(base, ROWS_PER_TILE)],
            nacc_hbm.at[c, pl.ds(base, ROWS_PER_TILE)],
        )
        pltpu.sync_copy(
            dsh.at[pl.ds(base, ROWS_PER_TILE)],
            dden_hbm.at[c, pl.ds(base, ROWS_PER_TILE)],
        )

    return edge_pass


def kernel(x, edge_index, W1, b1, att_src1, att_dst1, W2, b2, att_src2, att_dst2):
    n = x.shape[0]
    e = edge_index.shape[1]
    ne = e + n
    frac0 = 0.67
    cpt0 = int(frac0 * ne / (16 * CHUNK) + GRP) // GRP * GRP
    e0 = 16 * cpt0 * CHUNK
    cpt1 = -(-(ne - e0) // (16 * CHUNK))
    cpt1 = -(-cpt1 // GRP) * GRP
    cptm = max(cpt0, cpt1)

    loop = jnp.arange(n, dtype=edge_index.dtype)
    src = jnp.concatenate([edge_index[0], loop])
    dst = jnp.concatenate([edge_index[1], loop])

    def layout(arr):
        p0 = arr[:e0].reshape(16, cpt0, CHUNK)
        if cpt0 < cptm:
            p0 = jnp.pad(p0, ((0, 0), (0, cptm - cpt0), (0, 0)),
                         constant_values=n)
        p1 = jnp.pad(arr[e0:], (0, 16 * cpt1 * CHUNK - (ne - e0)),
                     constant_values=n).reshape(16, cpt1, CHUNK)
        if cpt1 < cptm:
            p1 = jnp.pad(p1, ((0, 0), (0, cptm - cpt1), (0, 0)),
                         constant_values=n)
        return jnp.concatenate([p0, p1], axis=0)

    src2d = layout(src)
    dst2d = layout(dst)

    perm = jnp.asarray(_PERM)
    xp = jnp.zeros((NP, D), f32).at[:n].set(x)
    W1p = W1[:, perm]
    W2p = W2[:, perm]
    attm1 = jnp.zeros((D, D), f32).at[:, 0].set(att_src1).at[:, 1].set(att_dst1)
    attm2 = jnp.zeros((D, D), f32).at[:, 0].set(att_src2).at[:, 1].set(att_dst2)
    attm1p = attm1[perm, :]
    attm2p = attm2[perm, :]
    b1_2d = b1.reshape(1, D)
    b2_2d = b2.reshape(1, D)

    edge_pass = _make_edge_pass(cpt0, cpt1)

    h1, aa1 = _mm_att(xp, W1p, attm1p)
    nacc1, dden1 = edge_pass(h1, aa1[:NA, 0], aa1[:NA, 1], src2d, dst2d)
    h2, aa2 = _combine_mm(nacc1, dden1, b1_2d, W2p, attm2p)
    nacc2, dden2 = edge_pass(h2, aa2[:NA, 0], aa2[:NA, 1], src2d, dst2d)
    outp = _combine_last(nacc2, dden2, b2_2d)
    return outp[:n]


# speedup vs baseline: 1.0492x; 1.0492x over previous
import functools

import jax
import jax.numpy as jnp
import numpy as np
from jax import lax
from jax.experimental import pallas as pl
from jax.experimental.pallas import tpu as pltpu
from jax.experimental.pallas import tpu_sc as plsc

N_NODES = 10000
NP = 10240
NA = 10112
D = 128
NTILES = 32
ROWS_PER_TILE = NP // 16
CHUNK = 128
GRP = 8
ROWBLK = 1280

f32 = jnp.float32
bf16 = jnp.bfloat16

_PERM = np.empty((D,), np.int64)
for _g in range(D // 32):
    for _j in range(16):
        _PERM[_g * 32 + 2 * _j] = _g * 32 + _j
        _PERM[_g * 32 + 2 * _j + 1] = _g * 32 + 16 + _j


def _mm_att_body(x_ref, w_ref, am_ref, h_ref, aa_ref):
    h = jnp.dot(x_ref[...], w_ref[...], preferred_element_type=f32)
    h_ref[...] = h.astype(bf16)
    aa_ref[...] = jnp.dot(h, am_ref[...], preferred_element_type=f32)


def _mm_att(xp, W, attm):
    grid = (NP // ROWBLK,)
    return pl.pallas_call(
        _mm_att_body,
        grid=grid,
        in_specs=[
            pl.BlockSpec((ROWBLK, D), lambda i: (i, 0)),
            pl.BlockSpec((D, D), lambda i: (0, 0)),
            pl.BlockSpec((D, D), lambda i: (0, 0)),
        ],
        out_specs=[
            pl.BlockSpec((ROWBLK, D), lambda i: (i, 0)),
            pl.BlockSpec((ROWBLK, D), lambda i: (i, 0)),
        ],
        out_shape=[
            jax.ShapeDtypeStruct((NP, D), bf16),
            jax.ShapeDtypeStruct((NP, D), f32),
        ],
    )(xp, W, attm)


def _combine_mm_body(n_ref, d_ref, b_ref, w_ref, am_ref, h_ref, aa_ref):
    num = n_ref[0] + n_ref[1]
    den = jnp.sum(d_ref[...], axis=0) + 1e-16
    o = num / den[:, None] + b_ref[...]
    h_in = jnp.maximum(o, 0.0)
    h = jnp.dot(h_in, w_ref[...], preferred_element_type=f32)
    h_ref[...] = h.astype(bf16)
    aa_ref[...] = jnp.dot(h, am_ref[...], preferred_element_type=f32)


def _combine_mm(nacc, dparts, b2d, W, attm):
    grid = (NP // ROWBLK,)
    return pl.pallas_call(
        _combine_mm_body,
        grid=grid,
        in_specs=[
            pl.BlockSpec((2, ROWBLK, D), lambda i: (0, i, 0)),
            pl.BlockSpec((2, ROWBLK), lambda i: (0, i)),
            pl.BlockSpec((1, D), lambda i: (0, 0)),
            pl.BlockSpec((D, D), lambda i: (0, 0)),
            pl.BlockSpec((D, D), lambda i: (0, 0)),
        ],
        out_specs=[
            pl.BlockSpec((ROWBLK, D), lambda i: (i, 0)),
            pl.BlockSpec((ROWBLK, D), lambda i: (i, 0)),
        ],
        out_shape=[
            jax.ShapeDtypeStruct((NP, D), bf16),
            jax.ShapeDtypeStruct((NP, D), f32),
        ],
    )(nacc, dparts, b2d, W, attm)


def _combine_last_body(n_ref, d_ref, b_ref, o_ref):
    num = n_ref[0] + n_ref[1]
    den = jnp.sum(d_ref[...], axis=0) + 1e-16
    o_ref[...] = num / den[:, None] + b_ref[...]


def _combine_last(nacc, dparts, b2d):
    grid = (NP // ROWBLK,)
    return pl.pallas_call(
        _combine_last_body,
        grid=grid,
        in_specs=[
            pl.BlockSpec((2, ROWBLK, D), lambda i: (0, i, 0)),
            pl.BlockSpec((2, ROWBLK), lambda i: (0, i)),
            pl.BlockSpec((1, D), lambda i: (0, 0)),
        ],
        out_specs=pl.BlockSpec((ROWBLK, D), lambda i: (i, 0)),
        out_shape=jax.ShapeDtypeStruct((NP, D), f32),
    )(nacc, dparts, b2d)


def _make_edge_pass(cpt0, cpt1):
    cpt = max(cpt0, cpt1)
    mesh = plsc.VectorSubcoreMesh(core_axis_name="c", subcore_axis_name="s")

    @functools.partial(
        pl.kernel,
        out_type=(
            jax.ShapeDtypeStruct((2, NP, D), f32),
            jax.ShapeDtypeStruct((2, NP), f32),
        ),
        mesh=mesh,
        compiler_params=pltpu.CompilerParams(
            needs_layout_passes=False, use_tc_tiling_on_sc=False),
        scratch_types=(
            pltpu.VMEM((2 * GRP, CHUNK), jnp.int32),
            pltpu.VMEM((2 * GRP, CHUNK), jnp.int32),
            pltpu.VMEM((2 * CHUNK, D), bf16),
            pltpu.VMEM((1, CHUNK), f32),
            pltpu.VMEM((1, CHUNK), f32),
            pltpu.VMEM((CHUNK, D), f32),
            pltpu.VMEM((CHUNK,), f32),
            pltpu.VMEM((ROWS_PER_TILE,), f32),
            pltpu.VMEM_SHARED((NP, D), f32),
            pltpu.VMEM_SHARED((NP,), f32),
            pltpu.VMEM_SHARED((NA,), f32),
            pltpu.VMEM_SHARED((NA,), f32),
            pltpu.SemaphoreType.DMA,
            pltpu.SemaphoreType.DMA,
        ),
    )
    def edge_pass(h_hbm, as_hbm, ad_hbm, src_hbm, dst_hbm,
                  nacc_hbm, dden_hbm,
                  srcl, dstl, rowsb, asr, adr, rows, epb, zb,
                  nsh, dsh, ash, adh, sem0, sem1):
        c = lax.axis_index("c")
        s = lax.axis_index("s")
        wid = c * 16 + s
        cpt_c = jnp.where(c == 0, cpt0, cpt1)
        ngrp = cpt_c // GRP

        na16 = NA // 16
        pltpu.sync_copy(as_hbm.at[pl.ds(s * na16, na16)],
                        ash.at[pl.ds(s * na16, na16)])
        pltpu.sync_copy(ad_hbm.at[pl.ds(s * na16, na16)],
                        adh.at[pl.ds(s * na16, na16)])

        zv = jnp.zeros((16,), f32)

        def zero_rows(r, _):
            for k in range(8):
                rows[r, pl.ds(16 * k, 16)] = zv
            return _
        lax.fori_loop(0, CHUNK, zero_rows, None)

        def zero_zb(i, _):
            zb[pl.ds(16 * i, 16)] = zv
            return _
        lax.fori_loop(0, ROWS_PER_TILE // 16, zero_zb, None)

        base = s * ROWS_PER_TILE
        for k in range(ROWS_PER_TILE // CHUNK):
            pltpu.sync_copy(rows, nsh.at[pl.ds(base + k * CHUNK, CHUNK)])
        pltpu.sync_copy(zb, dsh.at[pl.ds(base, ROWS_PER_TILE)])
        plsc.subcore_barrier()

        def stage_group(g):
            gp = lax.rem(g, 2)
            off = pl.multiple_of(g * GRP, GRP)
            dstrow = pl.multiple_of(gp * GRP, GRP)
            pltpu.sync_copy(src_hbm.at[wid, pl.ds(off, GRP)],
                            srcl.at[pl.ds(dstrow, GRP)])
            pltpu.sync_copy(dst_hbm.at[wid, pl.ds(off, GRP)],
                            dstl.at[pl.ds(dstrow, GRP)])

        def row_of(j):
            return lax.rem(j // GRP, 2) * GRP + lax.rem(j, GRP)

        def fire(j):
            p = lax.rem(j, 2)
            row = row_of(j)
            sm = [sem0, sem1]
            for pp in range(2):
                @pl.when(p == pp)
                def _():
                    off = pl.multiple_of(pp * CHUNK, CHUNK)
                    pltpu.async_copy(h_hbm.at[srcl.at[row]],
                                     rowsb.at[pl.ds(off, CHUNK)], sm[pp])

        def drain(j):
            p = lax.rem(j, 2)
            row = row_of(j)
            sm = [sem0, sem1]
            for pp in range(2):
                @pl.when(p == pp)
                def _():
                    off = pl.multiple_of(pp * CHUNK, CHUNK)
                    pltpu.make_async_copy(
                        h_hbm.at[srcl.at[row]],
                        rowsb.at[pl.ds(off, CHUNK)], sm[pp]).wait()

        stage_group(0)
        fire(0)

        def chunk_body(j, _):
            p = lax.rem(j, 2)
            row = row_of(j)

            g = j // GRP
            @pl.when(jnp.logical_and(lax.rem(j, GRP) == 0, g + 1 < ngrp))
            def _stage():
                stage_group(g + 1)

            drain(j)

            @pl.when(j + 1 < cpt_c)
            def _fire():
                fire(j + 1)

            pltpu.sync_copy(ash.at[srcl.at[row]], asr.at[0])
            pltpu.sync_copy(adh.at[dstl.at[row]], adr.at[0])

            for i in range(CHUNK // 16):
                a = asr[0, pl.ds(16 * i, 16)]
                b = adr[0, pl.ds(16 * i, 16)]
                e = a + b
                e = jnp.where(e >= 0.0, e, 0.2 * e)
                w = jnp.exp(e)
                epb[pl.ds(16 * i, 16)] = w

            pltpu.sync_copy(epb, dsh.at[dstl.at[row]], add=True)

            rb = p * CHUNK

            def scale16(i2, _):
                off = pl.multiple_of(i2 * 16, 16)
                wv = epb[pl.ds(off, 16)]
                r0 = rb + i2 * 16
                for l in range(16):
                    sc = wv[l]
                    for k in range(4):
                        packed = plsc.bitcast(
                            rowsb[r0 + l, pl.ds(32 * k, 32)], jnp.int32)
                        lo = plsc.bitcast(packed << 16, f32)
                        hi = plsc.bitcast(packed & jnp.int32(-65536), f32)
                        rows[i2 * 16 + l, pl.ds(32 * k, 16)] = lo * sc
                        rows[i2 * 16 + l, pl.ds(32 * k + 16, 16)] = hi * sc
                return _
            lax.fori_loop(0, CHUNK // 16, scale16, None)

            pltpu.sync_copy(rows, nsh.at[dstl.at[row]], add=True)
            return _

        lax.fori_loop(0, cpt_c, chunk_body, None)
        plsc.subcore_barrier()

        pltpu.sync_copy(
            nsh.at[pl.ds(base, ROWS_PER_TILE)],
            nacc_hbm.at[c, pl.ds(base, ROWS_PER_TILE)],
        )
        pltpu.sync_copy(
            dsh.at[pl.ds(base, ROWS_PER_TILE)],
            dden_hbm.at[c, pl.ds(base, ROWS_PER_TILE)],
        )

    return edge_pass


def kernel(x, edge_index, W1, b1, att_src1, att_dst1, W2, b2, att_src2, att_dst2):
    n = x.shape[0]
    e = edge_index.shape[1]
    ne = e + n
    frac0 = 0.64
    cpt0 = int(frac0 * ne / (16 * CHUNK) + GRP) // GRP * GRP
    e0 = 16 * cpt0 * CHUNK
    cpt1 = -(-(ne - e0) // (16 * CHUNK))
    cpt1 = -(-cpt1 // GRP) * GRP
    cptm = max(cpt0, cpt1)

    loop = jnp.arange(n, dtype=edge_index.dtype)
    src = jnp.concatenate([edge_index[0], loop])
    dst = jnp.concatenate([edge_index[1], loop])

    def layout(arr):
        p0 = arr[:e0].reshape(16, cpt0, CHUNK)
        if cpt0 < cptm:
            p0 = jnp.pad(p0, ((0, 0), (0, cptm - cpt0), (0, 0)),
                         constant_values=n)
        p1 = jnp.pad(arr[e0:], (0, 16 * cpt1 * CHUNK - (ne - e0)),
                     constant_values=n).reshape(16, cpt1, CHUNK)
        if cpt1 < cptm:
            p1 = jnp.pad(p1, ((0, 0), (0, cptm - cpt1), (0, 0)),
                         constant_values=n)
        return jnp.concatenate([p0, p1], axis=0)

    src2d = layout(src)
    dst2d = layout(dst)

    perm = jnp.asarray(_PERM)
    xp = jnp.zeros((NP, D), f32).at[:n].set(x)
    W1p = W1[:, perm]
    W2p = W2[:, perm]
    attm1 = jnp.zeros((D, D), f32).at[:, 0].set(att_src1).at[:, 1].set(att_dst1)
    attm2 = jnp.zeros((D, D), f32).at[:, 0].set(att_src2).at[:, 1].set(att_dst2)
    attm1p = attm1[perm, :]
    attm2p = attm2[perm, :]
    b1_2d = b1.reshape(1, D)
    b2_2d = b2.reshape(1, D)

    edge_pass = _make_edge_pass(cpt0, cpt1)

    h1, aa1 = _mm_att(xp, W1p, attm1p)
    nacc1, dden1 = edge_pass(h1, aa1[:NA, 0], aa1[:NA, 1], src2d, dst2d)
    h2, aa2 = _combine_mm(nacc1, dden1, b1_2d, W2p, attm2p)
    nacc2, dden2 = edge_pass(h2, aa2[:NA, 0], aa2[:NA, 1], src2d, dst2d)
    outp = _combine_last(nacc2, dden2, b2_2d)
    return outp[:n]
